# baseline (speedup 1.0000x reference)
import functools

import jax
import jax.numpy as jnp
from jax import lax
from jax.experimental import pallas as pl
from jax.experimental.pallas import tpu as pltpu
from jax.experimental.pallas import tpu_sc as plsc

N = 2048
ROWS = 256
NC, NS, L = 2, 16, 16
NW = NC * NS
RPW = ROWS // NW
EPS = 1e-8
DT = 1e-3
GROUPS = N // L
BS_STEPS = 12


def _prep_body(x_ref, y_ref, xt_ref, yt_ref):
    m = jnp.minimum(jnp.min(x_ref[...]), jnp.min(y_ref[...]))
    xt_ref[...] = jnp.transpose(x_ref[...], (0, 2, 1)) - m + EPS
    yt_ref[...] = jnp.transpose(y_ref[...], (0, 2, 1)) - m + EPS


_DNUMS = lax.GatherDimensionNumbers(
    offset_dims=(), collapsed_slice_dims=(0,), start_index_map=(0,)
)


def _vgather(v, idx):
    return lax.gather(
        v, idx[:, None], _DNUMS, slice_sizes=(1,),
        mode=lax.GatherScatterMode.PROMISE_IN_BOUNDS,
    )


def _bcast_last(v):
    return _vgather(v, jnp.full((L,), L - 1, jnp.int32))


def _cumsum_inplace(ref):
    def chunk2(i, carry):
        c0 = ref[pl.ds((2 * i) * L, L)]
        c1 = ref[pl.ds((2 * i + 1) * L, L)]
        s0 = plsc.cumsum(c0)
        s1 = plsc.cumsum(c1)
        cs0 = s0 + carry
        cs1 = s1 + _bcast_last(cs0)
        ref[pl.ds((2 * i) * L, L)] = cs0
        ref[pl.ds((2 * i + 1) * L, L)] = cs1
        return _bcast_last(cs1)
    return lax.fori_loop(0, GROUPS // 2, chunk2, jnp.zeros((L,), jnp.float32))


_SHIFT_IDX = None


def _direction_a(q_ref, t_ref, h_ref, r_qt, r_tq):
    lanes = lax.iota(jnp.int32, L)
    shift_idx = jnp.maximum(lanes - 1, 0)
    next_idx = jnp.minimum(lanes + 1, L - 1)

    def one_group(g, carry_q):
        q = q_ref[pl.ds(g * L, L)]
        qs = q * r_qt
        lo = jnp.zeros((L,), jnp.int32)
        hi = jnp.full((L,), N, jnp.int32)

        def step(_, lohi):
            lo, hi = lohi
            mid = lax.shift_right_logical(lo + hi, 1)
            tv = plsc.load_gather(t_ref, [jnp.minimum(mid, N - 1)])
            pred = tv < qs
            return jnp.where(pred, mid + 1, lo), jnp.where(pred, hi, mid)

        c, _ = lax.fori_loop(0, BS_STEPS, step, (lo, hi))
        ivec = g * L + lanes
        last_dup = jnp.logical_or(lanes == L - 1, c != _vgather(c, next_idx))
        plsc.store_scatter(
            h_ref, [jnp.minimum(c, N - 1)], ivec + 1,
            mask=jnp.logical_and(last_dup, c < N),
        )
        tprev = plsc.load_gather(t_ref, [jnp.maximum(c - 1, 0)])
        tprev = jnp.where(c > 0, tprev * r_tq, 0.0)
        qprev = _vgather(q, shift_idx)
        qprev = jnp.where(lanes == 0, carry_q, qprev)
        delta = q - jnp.maximum(qprev, tprev)
        di = (ivec - jnp.minimum(c, N - 1)).astype(jnp.float32)
        return delta * di * di, _bcast_last(q)

    def group2(i, state):
        acc0, acc1, carry_q = state
        w0, carry_q = one_group(2 * i, carry_q)
        w1, carry_q = one_group(2 * i + 1, carry_q)
        return acc0 + w0, acc1 + w1, carry_q

    z = jnp.zeros((L,), jnp.float32)
    acc0, acc1, _ = lax.fori_loop(0, GROUPS // 2, group2, (z, z, z))
    return acc0 + acc1


def _direction_b(q_ref, t_ref, h_ref, r_tq):
    lanes = lax.iota(jnp.int32, L)
    shift_idx = jnp.maximum(lanes - 1, 0)
    zi = jnp.zeros((L,), jnp.int32)

    def one_group(g, carry_h, carry_q):
        q = q_ref[pl.ds(g * L, L)]
        hm = jnp.maximum(plsc.cummax(h_ref[pl.ds(g * L, L)]), carry_h)
        h_ref[pl.ds(g * L, L)] = zi
        tprev = plsc.load_gather(t_ref, [jnp.maximum(hm - 1, 0)])
        tprev = jnp.where(hm > 0, tprev * r_tq, 0.0)
        qprev = _vgather(q, shift_idx)
        qprev = jnp.where(lanes == 0, carry_q, qprev)
        jvec = g * L + lanes
        delta = q - jnp.maximum(qprev, tprev)
        di = (jnp.minimum(hm, N - 1) - jvec).astype(jnp.float32)
        return delta * di * di, _bcast_last(hm), _bcast_last(q)

    def group2(i, state):
        acc0, acc1, carry_h, carry_q = state
        w0, carry_h, carry_q = one_group(2 * i, carry_h, carry_q)
        w1, carry_h, carry_q = one_group(2 * i + 1, carry_h, carry_q)
        return acc0 + w0, acc1 + w1, carry_h, carry_q

    z = jnp.zeros((L,), jnp.float32)
    acc0, acc1, _, _ = lax.fori_loop(0, GROUPS // 2, group2, (z, z, zi, z))
    return acc0 + acc1


def _sc_body(xt_hbm, yt_hbm, out_hbm, a_v, b_v, h_v, w_v):
    wid = lax.axis_index("s") * NC + lax.axis_index("c")
    lanes = lax.iota(jnp.int32, L)

    def zero_h(i, _):
        h_v[pl.ds(i * L, L)] = jnp.zeros((L,), jnp.int32)
        return 0
    lax.fori_loop(0, GROUPS, zero_h, 0)

    def row_body(r, wvec):
        row = wid * RPW + r
        pltpu.sync_copy(xt_hbm.at[pl.ds(row * N, N)], a_v)
        pltpu.sync_copy(yt_hbm.at[pl.ds(row * N, N)], b_v)
        ta_v = _cumsum_inplace(a_v)
        tb_v = _cumsum_inplace(b_v)
        r_ab = tb_v / ta_v
        r_ba = ta_v / tb_v
        acc = (_direction_a(a_v, b_v, h_v, r_ab, r_ba) / ta_v
               + _direction_b(b_v, a_v, h_v, r_ab) / tb_v)
        w = jnp.sum(acc) * (DT * DT)
        return jnp.where(lanes == r, w, wvec)

    wvec = lax.fori_loop(0, RPW, row_body, jnp.zeros((L,), jnp.float32))
    w_v[...] = wvec
    pltpu.sync_copy(w_v, out_hbm.at[wid])


def _sum_body(p_ref, o_ref):
    o_ref[...] = jnp.full((1, 1), jnp.sum(p_ref[...]), jnp.float32)


@jax.jit
def kernel(x, y):
    nb, nt, nr, nc = x.shape
    x2 = x.reshape(nb, nt, nr * nc)
    y2 = y.reshape(nb, nt, nr * nc)

    xt, yt = pl.pallas_call(
        _prep_body,
        out_shape=[
            jax.ShapeDtypeStruct((nb, nr * nc, nt), jnp.float32),
            jax.ShapeDtypeStruct((nb, nr * nc, nt), jnp.float32),
        ],
    )(x2, y2)

    mesh = plsc.VectorSubcoreMesh(
        core_axis_name="c", subcore_axis_name="s", num_cores=NC, num_subcores=NS
    )
    partials = pl.kernel(
        _sc_body,
        out_type=jax.ShapeDtypeStruct((NW, L), jnp.float32),
        mesh=mesh,
        compiler_params=pltpu.CompilerParams(needs_layout_passes=False),
        scratch_types=[
            pltpu.VMEM((N,), jnp.float32),
            pltpu.VMEM((N,), jnp.float32),
            pltpu.VMEM((N,), jnp.int32),
            pltpu.VMEM((L,), jnp.float32),
        ],
    )(xt.reshape(ROWS * N), yt.reshape(ROWS * N))

    loss = pl.pallas_call(
        _sum_body,
        out_shape=jax.ShapeDtypeStruct((1, 1), jnp.float32),
    )(partials.reshape(4, 128))
    return loss[0, 0]


# speedup vs baseline: 1.5443x; 1.5443x over previous
import functools

import jax
import jax.numpy as jnp
from jax import lax
from jax.experimental import pallas as pl
from jax.experimental.pallas import tpu as pltpu
from jax.experimental.pallas import tpu_sc as plsc

N = 2048
ROWS = 256
NC, NS, L = 2, 16, 16
NW = NC * NS
RPW = ROWS // NW
EPS = 1e-8
DT = 1e-3
GROUPS = N // L
BS_STEPS = 12


def _prep_body(x_ref, y_ref, xt_ref, yt_ref):
    m = jnp.minimum(jnp.min(x_ref[...]), jnp.min(y_ref[...]))
    xt_ref[...] = jnp.transpose(x_ref[...], (0, 2, 1)) - m + EPS
    yt_ref[...] = jnp.transpose(y_ref[...], (0, 2, 1)) - m + EPS


_DNUMS = lax.GatherDimensionNumbers(
    offset_dims=(), collapsed_slice_dims=(0,), start_index_map=(0,)
)


def _vgather(v, idx):
    return lax.gather(
        v, idx[:, None], _DNUMS, slice_sizes=(1,),
        mode=lax.GatherScatterMode.PROMISE_IN_BOUNDS,
    )


def _bcast_last(v):
    return _vgather(v, jnp.full((L,), L - 1, jnp.int32))


def _cumsum_inplace(ref):
    def chunk2(i, carry):
        c0 = ref[pl.ds((2 * i) * L, L)]
        c1 = ref[pl.ds((2 * i + 1) * L, L)]
        s0 = plsc.cumsum(c0)
        s1 = plsc.cumsum(c1)
        cs0 = s0 + carry
        cs1 = s1 + _bcast_last(cs0)
        ref[pl.ds((2 * i) * L, L)] = cs0
        ref[pl.ds((2 * i + 1) * L, L)] = cs1
        return _bcast_last(cs1)
    return lax.fori_loop(0, GROUPS // 2, chunk2, jnp.zeros((L,), jnp.float32))


_SHIFT_IDX = None


def _direction(q_ref, t_ref, r_qt, r_tq, right):
    lanes = lax.iota(jnp.int32, L)
    shift_idx = jnp.maximum(lanes - 1, 0)

    def one_group(g, carry_q):
        q = q_ref[pl.ds(g * L, L)]
        qs = q * r_qt
        lo = jnp.zeros((L,), jnp.int32)
        hi = jnp.full((L,), N, jnp.int32)

        def step(_, lohi):
            lo, hi = lohi
            mid = lax.shift_right_logical(lo + hi, 1)
            tv = plsc.load_gather(t_ref, [mid])
            pred = (tv <= qs) if right else (tv < qs)
            return jnp.where(pred, mid + 1, lo), jnp.where(pred, hi, mid)

        c, _ = lax.fori_loop(0, BS_STEPS, step, (lo, hi))
        tprev = plsc.load_gather(t_ref, [jnp.maximum(c - 1, 0)])
        tprev = jnp.where(c > 0, tprev * r_tq, 0.0)
        qprev = _vgather(q, shift_idx)
        qprev = jnp.where(lanes == 0, carry_q, qprev)
        ivec = g * L + lanes
        delta = q - jnp.maximum(qprev, tprev)
        di = (ivec - jnp.minimum(c, N - 1)).astype(jnp.float32)
        return delta * di * di, _bcast_last(q)

    def group2(i, state):
        acc0, acc1, carry_q = state
        w0, carry_q = one_group(2 * i, carry_q)
        w1, carry_q = one_group(2 * i + 1, carry_q)
        return acc0 + w0, acc1 + w1, carry_q

    z = jnp.zeros((L,), jnp.float32)
    acc0, acc1, _ = lax.fori_loop(0, GROUPS // 2, group2, (z, z, z))
    return acc0 + acc1


def _sc_body(xt_hbm, yt_hbm, out_hbm, a_v, b_v, w_v):
    wid = lax.axis_index("s") * NC + lax.axis_index("c")
    lanes = lax.iota(jnp.int32, L)

    def row_body(r, wvec):
        row = wid * RPW + r
        pltpu.sync_copy(xt_hbm.at[pl.ds(row * N, N)], a_v.at[pl.ds(0, N)])
        pltpu.sync_copy(yt_hbm.at[pl.ds(row * N, N)], b_v.at[pl.ds(0, N)])
        ta_v = _cumsum_inplace(a_v)
        tb_v = _cumsum_inplace(b_v)
        inf = jnp.full((L,), jnp.inf, jnp.float32)
        a_v[pl.ds(N, L)] = inf
        b_v[pl.ds(N, L)] = inf
        r_ab = tb_v / ta_v
        r_ba = ta_v / tb_v
        acc = (_direction(a_v, b_v, r_ab, r_ba, right=False) / ta_v
               + _direction(b_v, a_v, r_ba, r_ab, right=True) / tb_v)
        w = jnp.sum(acc) * (DT * DT)
        return jnp.where(lanes == r, w, wvec)

    wvec = lax.fori_loop(0, RPW, row_body, jnp.zeros((L,), jnp.float32))
    w_v[...] = wvec
    pltpu.sync_copy(w_v, out_hbm.at[wid])


def _sum_body(p_ref, o_ref):
    o_ref[...] = jnp.full((1, 1), jnp.sum(p_ref[...]), jnp.float32)


@jax.jit
def kernel(x, y):
    nb, nt, nr, nc = x.shape
    x2 = x.reshape(nb, nt, nr * nc)
    y2 = y.reshape(nb, nt, nr * nc)

    xt, yt = pl.pallas_call(
        _prep_body,
        out_shape=[
            jax.ShapeDtypeStruct((nb, nr * nc, nt), jnp.float32),
            jax.ShapeDtypeStruct((nb, nr * nc, nt), jnp.float32),
        ],
    )(x2, y2)

    mesh = plsc.VectorSubcoreMesh(
        core_axis_name="c", subcore_axis_name="s", num_cores=NC, num_subcores=NS
    )
    partials = pl.kernel(
        _sc_body,
        out_type=jax.ShapeDtypeStruct((NW, L), jnp.float32),
        mesh=mesh,
        compiler_params=pltpu.CompilerParams(needs_layout_passes=False),
        scratch_types=[
            pltpu.VMEM((N + L,), jnp.float32),
            pltpu.VMEM((N + L,), jnp.float32),
            pltpu.VMEM((L,), jnp.float32),
        ],
    )(xt.reshape(ROWS * N), yt.reshape(ROWS * N))

    loss = pl.pallas_call(
        _sum_body,
        out_shape=jax.ShapeDtypeStruct((1, 1), jnp.float32),
    )(partials.reshape(4, 128))
    return loss[0, 0]


# speedup vs baseline: 1.6329x; 1.0574x over previous
import functools

import jax
import jax.numpy as jnp
from jax import lax
from jax.experimental import pallas as pl
from jax.experimental.pallas import tpu as pltpu
from jax.experimental.pallas import tpu_sc as plsc

N = 2048
ROWS = 256
NC, NS, L = 2, 16, 16
NW = NC * NS
RPW = ROWS // NW
EPS = 1e-8
DT = 1e-3
GROUPS = N // L
BS_STEPS = 12


def _prep_body(x_ref, y_ref, xt_ref, yt_ref):
    m = jnp.minimum(jnp.min(x_ref[...]), jnp.min(y_ref[...]))
    xt_ref[...] = jnp.transpose(x_ref[...], (0, 2, 1)) - m + EPS
    yt_ref[...] = jnp.transpose(y_ref[...], (0, 2, 1)) - m + EPS


_DNUMS = lax.GatherDimensionNumbers(
    offset_dims=(), collapsed_slice_dims=(0,), start_index_map=(0,)
)


def _vgather(v, idx):
    return lax.gather(
        v, idx[:, None], _DNUMS, slice_sizes=(1,),
        mode=lax.GatherScatterMode.PROMISE_IN_BOUNDS,
    )


def _bcast_last(v):
    return _vgather(v, jnp.full((L,), L - 1, jnp.int32))


def _cumsum_inplace(ref):
    def chunk2(i, carry):
        c0 = ref[pl.ds((2 * i) * L, L)]
        c1 = ref[pl.ds((2 * i + 1) * L, L)]
        s0 = plsc.cumsum(c0)
        s1 = plsc.cumsum(c1)
        cs0 = s0 + carry
        cs1 = s1 + _bcast_last(cs0)
        ref[pl.ds((2 * i) * L, L)] = cs0
        ref[pl.ds((2 * i + 1) * L, L)] = cs1
        return _bcast_last(cs1)
    return lax.fori_loop(0, GROUPS // 2, chunk2, jnp.zeros((L,), jnp.float32))


_SHIFT_IDX = None


def _direction(q_ref, t_ref, r_qt, r_tq, right):
    lanes = lax.iota(jnp.int32, L)
    shift_idx = jnp.maximum(lanes - 1, 0)

    def one_group(g, carry_q):
        q = q_ref[pl.ds(g * L, L)]
        qs = q * r_qt
        lo = jnp.zeros((L,), jnp.int32)
        hi = jnp.full((L,), N, jnp.int32)

        def step(_, lohi):
            lo, hi = lohi
            mid = lax.shift_right_logical(lo + hi, 1)
            tv = plsc.load_gather(t_ref, [mid])
            pred = (tv <= qs) if right else (tv < qs)
            return jnp.where(pred, mid + 1, lo), jnp.where(pred, hi, mid)

        c, _ = lax.fori_loop(0, BS_STEPS, step, (lo, hi))
        tprev = plsc.load_gather(t_ref, [jnp.maximum(c - 1, 0)])
        tprev = jnp.where(c > 0, tprev * r_tq, 0.0)
        qprev = _vgather(q, shift_idx)
        qprev = jnp.where(lanes == 0, carry_q, qprev)
        ivec = g * L + lanes
        delta = q - jnp.maximum(qprev, tprev)
        di = (ivec - jnp.minimum(c, N - 1)).astype(jnp.float32)
        return delta * di * di, _bcast_last(q)

    def group2(i, state):
        acc0, acc1, carry_q = state
        w0, carry_q = one_group(2 * i, carry_q)
        w1, carry_q = one_group(2 * i + 1, carry_q)
        return acc0 + w0, acc1 + w1, carry_q

    z = jnp.zeros((L,), jnp.float32)
    acc0, acc1, _ = lax.fori_loop(0, GROUPS // 2, group2, (z, z, z))
    return acc0 + acc1


def _sc_body(xt_hbm, yt_hbm, out_hbm, a0_v, b0_v, a1_v, b1_v, w_v, sem0, sem1):
    wid = lax.axis_index("s") * NC + lax.axis_index("c")
    lanes = lax.iota(jnp.int32, L)
    base = wid * RPW

    def start(r, a_v, b_v, sem):
        pltpu.async_copy(
            xt_hbm.at[pl.ds((base + r) * N, N)], a_v.at[pl.ds(0, N)], sem)
        pltpu.async_copy(
            yt_hbm.at[pl.ds((base + r) * N, N)], b_v.at[pl.ds(0, N)], sem)

    def wait(r, a_v, b_v, sem):
        pltpu.make_async_copy(
            xt_hbm.at[pl.ds((base + r) * N, N)], a_v.at[pl.ds(0, N)], sem
        ).wait()
        pltpu.make_async_copy(
            yt_hbm.at[pl.ds((base + r) * N, N)], b_v.at[pl.ds(0, N)], sem
        ).wait()

    def compute(r, a_v, b_v, wvec):
        ta_v = _cumsum_inplace(a_v)
        tb_v = _cumsum_inplace(b_v)
        inf = jnp.full((L,), jnp.inf, jnp.float32)
        a_v[pl.ds(N, L)] = inf
        b_v[pl.ds(N, L)] = inf
        r_ab = tb_v / ta_v
        r_ba = ta_v / tb_v
        acc = (_direction(a_v, b_v, r_ab, r_ba, right=False) / ta_v
               + _direction(b_v, a_v, r_ba, r_ab, right=True) / tb_v)
        w = jnp.sum(acc) * (DT * DT)
        return jnp.where(lanes == r, w, wvec)

    start(0, a0_v, b0_v, sem0)

    def row_pair(i, wvec):
        r0 = 2 * i
        r1 = 2 * i + 1
        start(r1, a1_v, b1_v, sem1)
        wait(r0, a0_v, b0_v, sem0)
        wvec = compute(r0, a0_v, b0_v, wvec)

        @pl.when(i < RPW // 2 - 1)
        def _():
            start(r0 + 2, a0_v, b0_v, sem0)

        wait(r1, a1_v, b1_v, sem1)
        return compute(r1, a1_v, b1_v, wvec)

    wvec = lax.fori_loop(0, RPW // 2, row_pair, jnp.zeros((L,), jnp.float32))
    w_v[...] = wvec
    pltpu.sync_copy(w_v, out_hbm.at[wid])


def _sum_body(p_ref, o_ref):
    o_ref[...] = jnp.full((1, 1), jnp.sum(p_ref[...]), jnp.float32)


@jax.jit
def kernel(x, y):
    nb, nt, nr, nc = x.shape
    x2 = x.reshape(nb, nt, nr * nc)
    y2 = y.reshape(nb, nt, nr * nc)

    xt, yt = pl.pallas_call(
        _prep_body,
        out_shape=[
            jax.ShapeDtypeStruct((nb, nr * nc, nt), jnp.float32),
            jax.ShapeDtypeStruct((nb, nr * nc, nt), jnp.float32),
        ],
    )(x2, y2)

    mesh = plsc.VectorSubcoreMesh(
        core_axis_name="c", subcore_axis_name="s", num_cores=NC, num_subcores=NS
    )
    partials = pl.kernel(
        _sc_body,
        out_type=jax.ShapeDtypeStruct((NW, L), jnp.float32),
        mesh=mesh,
        compiler_params=pltpu.CompilerParams(needs_layout_passes=False),
        scratch_types=[
            pltpu.VMEM((N + L,), jnp.float32),
            pltpu.VMEM((N + L,), jnp.float32),
            pltpu.VMEM((N + L,), jnp.float32),
            pltpu.VMEM((N + L,), jnp.float32),
            pltpu.VMEM((L,), jnp.float32),
            pltpu.SemaphoreType.DMA,
            pltpu.SemaphoreType.DMA,
        ],
    )(xt.reshape(ROWS * N), yt.reshape(ROWS * N))

    loss = pl.pallas_call(
        _sum_body,
        out_shape=jax.ShapeDtypeStruct((1, 1), jnp.float32),
    )(partials.reshape(4, 128))
    return loss[0, 0]
